# Initial kernel scaffold; baseline (speedup 1.0000x reference)
import jax
import jax.numpy as jnp
from jax.experimental import pallas as pl


def kernel(support_tree_emb, support_rel, query_head, cos_rel_all, t_h, Train, rel_emb_table, edge_matrix, w_ih, w_hh, b_ih, b_hh):
    raise NotImplementedError("write your pallas kernel here")


# speedup vs baseline: 8.8292x; 8.8292x over previous
import functools

import jax
import jax.numpy as jnp
from jax import lax
from jax.experimental import pallas as pl
from jax.experimental.pallas import tpu as pltpu
from jax.experimental.pallas import tpu_sc as plsc

D = 128
NEI = 32
K = 16
B = 1024
R = 2000
NE = 50000
S = 16
RP = 2048
EW = 128
NC = 2
NS = 16
NW = NC * NS
LANES = 16


def _mesh():
    return plsc.VectorSubcoreMesh(core_axis_name="c", subcore_axis_name="s")


def _wid():
    return lax.axis_index("s") * NC + lax.axis_index("c")


def _dg16(vec, idx):
    return lax.gather(
        vec, idx[:, None],
        lax.GatherDimensionNumbers(
            offset_dims=(), collapsed_slice_dims=(0,), start_index_map=(0,)),
        (1,), mode=lax.GatherScatterMode.PROMISE_IN_BOUNDS)


def _sc_rel_score(cos_pad, srel):
    @functools.partial(
        pl.kernel,
        out_type=jax.ShapeDtypeStruct((RP,), jnp.float32),
        mesh=_mesh(),
        compiler_params=pltpu.CompilerParams(needs_layout_passes=False),
        scratch_types=[
            pltpu.VMEM((S,), jnp.int32),
            pltpu.VMEM((S, RP), jnp.float32),
            pltpu.VMEM((RP,), jnp.float32),
            pltpu.SemaphoreType.DMA,
        ],
    )
    def k(cos_hbm, srel_hbm, out_hbm, idx_v, rows_v, acc_v, sem):
        @pl.when(_wid() == 0)
        def _():
            pltpu.sync_copy(srel_hbm, idx_v)
            pltpu.async_copy(cos_hbm.at[idx_v], rows_v, sem).wait()

            def body(j, carry):
                sl = pl.ds(j * LANES, LANES)
                m = rows_v[0, sl]
                for s in range(1, S):
                    m = jnp.maximum(m, rows_v[s, sl])
                acc_v[sl] = m
                return carry

            lax.fori_loop(0, RP // LANES, body, 0)
            pltpu.sync_copy(acc_v, out_hbm)

    return k(cos_pad, srel)


def _make_edge_gather(C):
    WB = B // NW
    NI = WB * C
    CI = min(128, NI)
    NCH = NI // CI
    OUTN = B * C * NEI

    @functools.partial(
        pl.kernel,
        out_type=(
            jax.ShapeDtypeStruct((OUTN,), jnp.float32),
            jax.ShapeDtypeStruct((OUTN,), jnp.int32),
            jax.ShapeDtypeStruct((OUTN,), jnp.int32),
        ),
        mesh=_mesh(),
        compiler_params=pltpu.CompilerParams(needs_layout_passes=False),
        scratch_types=[
            pltpu.VMEM((RP,), jnp.float32),
            pltpu.VMEM((CI,), jnp.int32),
            pltpu.VMEM((CI, EW), jnp.int32),
            pltpu.VMEM((CI * NEI,), jnp.float32),
            pltpu.VMEM((CI * NEI,), jnp.int32),
            pltpu.VMEM((CI * NEI,), jnp.int32),
            pltpu.SemaphoreType.DMA,
        ],
    )
    def k(edge_hbm, cur_hbm, rs_hbm, osc_hbm, oent_hbm, orel_hbm,
          tab_v, idx_v, rows_v, osc_v, oent_v, orel_v, sem):
        wid = _wid()
        pltpu.sync_copy(rs_hbm, tab_v)
        lane = lax.iota(jnp.int32, LANES)
        pat_e = (lane & 7) * 2
        pat_o = pat_e + 1
        lo = lane < 8
        for ch in range(NCH):
            off = wid * NI + ch * CI
            pltpu.sync_copy(cur_hbm.at[pl.ds(off, CI)], idx_v)
            pltpu.async_copy(edge_hbm.at[idx_v], rows_v, sem).wait()

            def body(r, carry):
                for v2 in range(2):
                    a = rows_v[r, pl.ds(v2 * 2 * LANES, LANES)]
                    b = rows_v[r, pl.ds(v2 * 2 * LANES + LANES, LANES)]
                    entv = jnp.where(lo, _dg16(a, pat_e), _dg16(b, pat_e))
                    relv = jnp.where(lo, _dg16(a, pat_o), _dg16(b, pat_o))
                    scv = plsc.load_gather(tab_v, [relv])
                    o = pl.ds(r * NEI + v2 * LANES, LANES)
                    osc_v[o] = scv
                    oent_v[o] = entv
                    orel_v[o] = relv
                return carry

            lax.fori_loop(0, CI, body, 0)
            ooff = off * NEI
            pltpu.sync_copy(osc_v, osc_hbm.at[pl.ds(ooff, CI * NEI)])
            pltpu.sync_copy(oent_v, oent_hbm.at[pl.ds(ooff, CI * NEI)])
            pltpu.sync_copy(orel_v, orel_hbm.at[pl.ds(ooff, CI * NEI)])

    return k


_edge_gather_1 = _make_edge_gather(1)
_edge_gather_16 = _make_edge_gather(K)


def _sc_emb_gather(tab, idx_flat):
    NI = (B * K) // NW
    CI = 128
    NCH = NI // CI

    @functools.partial(
        pl.kernel,
        out_type=jax.ShapeDtypeStruct((B * K, D), jnp.float32),
        mesh=_mesh(),
        compiler_params=pltpu.CompilerParams(needs_layout_passes=False),
        scratch_types=[
            pltpu.VMEM((CI,), jnp.int32),
            pltpu.VMEM((CI, D), jnp.float32),
            pltpu.SemaphoreType.DMA,
        ],
    )
    def k(tab_hbm, idx_hbm, out_hbm, idx_v, rows_v, sem):
        wid = _wid()
        for ch in range(NCH):
            off = wid * NI + ch * CI
            pltpu.sync_copy(idx_hbm.at[pl.ds(off, CI)], idx_v)
            pltpu.async_copy(tab_hbm.at[idx_v], rows_v, sem).wait()
            pltpu.sync_copy(rows_v, out_hbm.at[pl.ds(off, CI)])

    return k(tab, idx_flat)


def _make_topk(N, with_prev):
    Bb = 128

    def body(sc_ref, ent_ref, rel_ref, *rest):
        if with_prev:
            pent_ref, prel_ref, aent_ref, arel_ref, pf_ref, pn_ref, arp_ref = rest
        else:
            aent_ref, arel_ref = rest
        sc = sc_ref[...]
        ent = ent_ref[...]
        rel = rel_ref[...]
        colid = lax.broadcasted_iota(jnp.int32, (Bb, N), 1)
        if with_prev:
            pent = pent_ref[...]
            prel = prel_ref[...]
            jid = lax.broadcasted_iota(jnp.int32, (Bb, K), 1)
        aent_c, arel_c, pf_c, pn_c, arp_c = [], [], [], [], []
        for _ in range(K):
            m = jnp.max(sc, axis=1, keepdims=True)
            eq = sc == m
            idx = jnp.min(jnp.where(eq, colid, N), axis=1, keepdims=True)
            hit = colid == idx
            aent_c.append(jnp.sum(jnp.where(hit, ent, 0), axis=1, keepdims=True))
            arel_c.append(jnp.sum(jnp.where(hit, rel, 0), axis=1, keepdims=True))
            sc = jnp.where(hit, -1.0, sc)
            if with_prev:
                p = idx // NEI
                pf_c.append(p.astype(jnp.float32))
                hp = jid == p
                pn_c.append(jnp.sum(jnp.where(hp, pent, 0), axis=1, keepdims=True))
                arp_c.append(jnp.sum(jnp.where(hp, prel, 0), axis=1, keepdims=True))
        aent_ref[...] = jnp.concatenate(aent_c, axis=1)
        arel_ref[...] = jnp.concatenate(arel_c, axis=1)
        if with_prev:
            pf_ref[...] = jnp.concatenate(pf_c, axis=1)
            pn_ref[...] = jnp.concatenate(pn_c, axis=1)
            arp_ref[...] = jnp.concatenate(arp_c, axis=1)

    grid = (B // Bb,)
    bigspec = pl.BlockSpec((Bb, N), lambda i: (i, 0))
    kspec = pl.BlockSpec((Bb, K), lambda i: (i, 0))
    in_specs = [bigspec, bigspec, bigspec] + ([kspec, kspec] if with_prev else [])
    n_out = 5 if with_prev else 2
    out_shape = tuple(
        jax.ShapeDtypeStruct((B, K), jnp.float32 if j == 2 else jnp.int32)
        for j in range(n_out)
    )
    return pl.pallas_call(
        body,
        grid=grid,
        in_specs=in_specs,
        out_specs=tuple([kspec] * n_out),
        out_shape=out_shape,
    )


_topk_1 = _make_topk(NEI, False)
_topk_16 = _make_topk(K * NEI, True)


def _make_gru(with_h):
    Mb = 2048
    GB = Mb // K

    def body(*refs):
        if with_h:
            (x_ref, wih_ref, whh_ref, bih_ref, bhh_ref, pe_ref, p_ref,
             out_ref) = refs
        else:
            x_ref, wih_ref, whh_ref, bih_ref, bhh_ref, out_ref = refs
        x = x_ref[...]
        gi = lax.dot_general(x, wih_ref[...], (((1,), (1,)), ((), ())),
                             precision=lax.Precision.HIGHEST,
                             preferred_element_type=jnp.float32)
        gi = gi + bih_ref[...]
        i_r = gi[:, :D]
        i_z = gi[:, D:2 * D]
        i_n = gi[:, 2 * D:]
        if with_h:
            pe = pe_ref[...]
            pe3 = pe.reshape(GB, K, D)
            p1 = p_ref[...]
            h = jnp.zeros((Mb, D), jnp.float32)
            for j in range(K):
                src = lax.broadcast_in_dim(
                    pe3[:, j, :], (GB, K, D), (0, 2)).reshape(Mb, D)
                h = jnp.where(p1 == j, src, h)
            gh = lax.dot_general(h, whh_ref[...], (((1,), (1,)), ((), ())),
                                 precision=lax.Precision.HIGHEST,
                                 preferred_element_type=jnp.float32)
            gh = gh + bhh_ref[...]
            h_r = gh[:, :D]
            h_z = gh[:, D:2 * D]
            h_n = gh[:, 2 * D:]
        else:
            bhh = bhh_ref[...]
            h_r = bhh[:, :D]
            h_z = bhh[:, D:2 * D]
            h_n = bhh[:, 2 * D:]
        r = 1.0 / (1.0 + jnp.exp(-(i_r + h_r)))
        z = 1.0 / (1.0 + jnp.exp(-(i_z + h_z)))
        n = jnp.tanh(i_n + r * h_n)
        if with_h:
            out_ref[...] = (1.0 - z) * n + z * h
        else:
            out_ref[...] = (1.0 - z) * n

    grid = ((B * K) // Mb,)
    xspec = pl.BlockSpec((Mb, D), lambda i: (i, 0))
    wspec = pl.BlockSpec((3 * D, D), lambda i: (0, 0))
    bspec = pl.BlockSpec((1, 3 * D), lambda i: (0, 0))
    in_specs = [xspec, wspec, wspec, bspec, bspec]
    if with_h:
        in_specs += [xspec, pl.BlockSpec((Mb, 1), lambda i: (i, 0))]
    return pl.pallas_call(
        body,
        grid=grid,
        in_specs=in_specs,
        out_specs=xspec,
        out_shape=jax.ShapeDtypeStruct((B * K, D), jnp.float32),
    )


_gru_0 = _make_gru(False)
_gru_h = _make_gru(True)


def kernel(support_tree_emb, support_rel, query_head, cos_rel_all, t_h, Train,
           rel_emb_table, edge_matrix, w_ih, w_hh, b_ih, b_hh):
    srel = support_rel.reshape(-1).astype(jnp.int32)
    qh = query_head.astype(jnp.int32)
    cos_pad = jnp.pad(cos_rel_all, ((0, 0), (0, RP - R)))
    edge2d = jnp.pad(edge_matrix.reshape(NE, 2 * NEI),
                     ((0, 0), (0, EW - 2 * NEI)))
    bih2 = b_ih.reshape(1, 3 * D)
    bhh2 = b_hh.reshape(1, 3 * D)

    rel_score = _sc_rel_score(cos_pad, srel)

    scf, entf, relf = _edge_gather_1(edge2d, qh, rel_score)
    aim_ent1, aim_rel1 = _topk_1(
        scf.reshape(B, NEI), entf.reshape(B, NEI), relf.reshape(B, NEI))
    rel_e1 = _sc_emb_gather(rel_emb_table, aim_rel1.reshape(-1))
    emb1 = _gru_0(rel_e1, w_ih, w_hh, bih2, bhh2)

    def hop(aim_ent_p, aim_rel_p, emb_p):
        scf, entf, relf = _edge_gather_16(edge2d, aim_ent_p.reshape(-1), rel_score)
        aent, arel, pf, pn, arp = _topk_16(
            scf.reshape(B, K * NEI), entf.reshape(B, K * NEI),
            relf.reshape(B, K * NEI), aim_ent_p, aim_rel_p)
        rel_e = _sc_emb_gather(rel_emb_table, arel.reshape(-1))
        emb = _gru_h(rel_e, w_ih, w_hh, bih2, bhh2, emb_p,
                     pf.astype(jnp.int32).reshape(B * K, 1))
        return aent, arel, emb, pf, pn, arp

    aim_ent2, aim_rel2, emb2, pf2, pn2, arp2 = hop(aim_ent1, aim_rel1, emb1)
    aim_ent3, aim_rel3, emb3, pf3, pn3, arp3 = hop(aim_ent2, aim_rel2, emb2)

    tree_node = jnp.stack([aim_ent1, aim_ent2, aim_ent3], 1)
    tree_emb_all = jnp.stack(
        [emb1.reshape(B, K, D), emb2.reshape(B, K, D), emb3.reshape(B, K, D)], 1)
    parent_index = jnp.stack(
        [pf2, pf3, jnp.tile(jnp.arange(K, dtype=jnp.float32)[None, :], (B, 1))], 1)
    parent_node = jnp.stack([jnp.tile(qh[:, None], (1, K)), pn2, pn3], 1)
    aim_rel_all = jnp.stack([arp2, arp3, aim_rel3], 1)
    return tree_node, tree_emb_all, parent_index, parent_node, aim_rel_all
